# baseline (speedup 1.0000x reference)
import functools

import jax
import jax.numpy as jnp
from jax.experimental import pallas as pl

MAP_RES = 16.0
K_TOP = 8
TQ = 512
TK = 2048
TQ2 = 512
BIG_I = 2**30


def _p1_body(k_ref, q_ref, v_ref, i_ref, *, k_real, tk):
    ki = pl.program_id(0)
    kb = k_ref[...]
    q = q_ref[...]
    kn = jnp.sum(kb * kb, axis=1, keepdims=True)
    qn = jax.lax.dot_general(jnp.ones((1, q.shape[1]), jnp.float32), q * q,
                             (((1,), (1,)), ((), ())),
                             preferred_element_type=jnp.float32)
    dot = jax.lax.dot_general(kb, q, (((1,), (1,)), ((), ())),
                              preferred_element_type=jnp.float32)
    d2 = (qn + kn) - 2.0 * dot
    rloc = jax.lax.broadcasted_iota(jnp.int32, d2.shape, 0)
    d2 = jnp.where(rloc < k_real - ki * tk, d2, jnp.inf)
    row_f = rloc.astype(jnp.float32)
    r1 = jax.lax.broadcasted_iota(jnp.int32, (1, tk), 1).astype(jnp.float32)
    w = jnp.concatenate([jnp.ones_like(r1), r1, r1 * r1], axis=0)
    vs, ids = [], []
    for _ in range(K_TOP):
        m = jnp.min(d2, axis=0, keepdims=True)
        bits = d2 == m
        onehot = jnp.where(bits, 1.0, 0.0)
        mom = jax.lax.dot_general(w, onehot, (((1,), (0,)), ((), ())),
                                  precision=jax.lax.Precision.HIGHEST,
                                  preferred_element_type=jnp.float32)
        cnt, s, ss = mom[0:1], mom[1:2], mom[2:3]
        delta = jnp.maximum(2.0 * ss - s * s, 0.0)
        am = jnp.round(jnp.where(cnt > 1.5, 0.5 * (s - jnp.sqrt(delta)), s))
        vs.append(m)
        ids.append(am)
        d2 = jnp.where(row_f == am, jnp.inf, d2)
    v_ref[...] = jnp.concatenate(vs, axis=0)
    i_ref[...] = (jnp.concatenate(ids, axis=0)
                  + (ki * tk).astype(jnp.float32)).astype(jnp.int32)


def _p2_body(v_ref, i_ref, qi_ref, kf_ref, td_ref, *, tq2):
    qb = pl.program_id(0)
    v = v_ref[...]
    idx = i_ref[...]
    tds, kfs = [], []
    for _ in range(K_TOP):
        m = jnp.min(v, axis=0, keepdims=True)
        am = jnp.min(jnp.where(v == m, idx, BIG_I), axis=0,
                     keepdims=True)
        tds.append(m)
        kfs.append(am)
        v = jnp.where(idx == am, jnp.inf, v)
    d2t = jnp.concatenate(tds, axis=0)
    kf = jnp.concatenate(kfs, axis=0)
    td = jnp.sqrt(jnp.maximum(d2t, 1e-12))
    valid = td <= MAP_RES
    rows = qb * tq2 + jax.lax.broadcasted_iota(jnp.int32, td.shape, 1)
    qi_ref[...] = jnp.where(valid, rows, -1)
    kf_ref[...] = jnp.where(valid, kf, -1)
    td_ref[...] = td


def kernel(queries, keys, k):
    del k
    nq, d = queries.shape
    k_real = keys.shape[0]
    kp = ((k_real + TK - 1) // TK) * TK
    nb = kp // TK
    keys_p = jnp.pad(keys, ((0, kp - k_real), (0, 0)))

    p1 = pl.pallas_call(
        functools.partial(_p1_body, k_real=k_real, tk=TK),
        grid=(nb, nq // TQ),
        in_specs=[
            pl.BlockSpec((TK, d), lambda ki, qi: (ki, 0)),
            pl.BlockSpec((TQ, d), lambda ki, qi: (qi, 0)),
        ],
        out_specs=[
            pl.BlockSpec((K_TOP, TQ), lambda ki, qi: (ki, qi)),
            pl.BlockSpec((K_TOP, TQ), lambda ki, qi: (ki, qi)),
        ],
        out_shape=[
            jax.ShapeDtypeStruct((nb * K_TOP, nq), jnp.float32),
            jax.ShapeDtypeStruct((nb * K_TOP, nq), jnp.int32),
        ],
    )
    pv, pi = p1(keys_p, queries)

    p2 = pl.pallas_call(
        functools.partial(_p2_body, tq2=TQ2),
        grid=(nq // TQ2,),
        in_specs=[
            pl.BlockSpec((nb * K_TOP, TQ2), lambda qb: (0, qb)),
            pl.BlockSpec((nb * K_TOP, TQ2), lambda qb: (0, qb)),
        ],
        out_specs=[
            pl.BlockSpec((K_TOP, TQ2), lambda qb: (0, qb)),
            pl.BlockSpec((K_TOP, TQ2), lambda qb: (0, qb)),
            pl.BlockSpec((K_TOP, TQ2), lambda qb: (0, qb)),
        ],
        out_shape=[
            jax.ShapeDtypeStruct((K_TOP, nq), jnp.int32),
            jax.ShapeDtypeStruct((K_TOP, nq), jnp.int32),
            jax.ShapeDtypeStruct((K_TOP, nq), jnp.float32),
        ],
    )
    qi_t, kf_t, td_t = p2(pv, pi)
    return qi_t.T, kf_t.T, td_t.T


# speedup vs baseline: 4.6205x; 4.6205x over previous
import functools

import jax
import jax.numpy as jnp
from jax.experimental import pallas as pl

MAP_RES = 16.0
K_TOP = 8
TQ = 512
TK = 2048
TQ2 = 512
BIG_I = 2**30


def _p1_body(k_ref, q_ref, v_ref, i_ref, *, k_real, tk):
    ki = pl.program_id(0)
    kb = k_ref[...]
    q = q_ref[...]
    kn = jnp.sum(kb * kb, axis=1, keepdims=True)
    qn = jax.lax.dot_general(jnp.ones((1, q.shape[1]), jnp.float32), q * q,
                             (((1,), (1,)), ((), ())),
                             preferred_element_type=jnp.float32)
    dot = jax.lax.dot_general(kb, q, (((1,), (1,)), ((), ())),
                              preferred_element_type=jnp.float32)
    d2 = (qn + kn) - 2.0 * dot
    rloc = jax.lax.broadcasted_iota(jnp.int32, d2.shape, 0)
    d2 = jnp.where(rloc < k_real - ki * tk, d2, jnp.inf)
    r1 = jax.lax.broadcasted_iota(jnp.int32, (1, tk), 1)
    r2i = r1 * r1
    w = jnp.concatenate([
        jnp.ones((1, tk), jnp.int32),
        (r1 >> 3) << 3, r1 & 7,
        (r2i >> 16) << 16, ((r2i >> 8) & 255) << 8, r2i & 255,
    ], axis=0).astype(jnp.float32)
    ms, moms = [], []
    for _ in range(K_TOP):
        m = jnp.min(d2, axis=0, keepdims=True)
        bits = d2 == m
        onehot = jnp.where(bits, 1.0, 0.0)
        moms.append(jax.lax.dot_general(w, onehot, (((1,), (0,)), ((), ())),
                                        preferred_element_type=jnp.float32))
        ms.append(m)
        d2 = jnp.where(bits, jnp.inf, d2)
    mv = jnp.concatenate(ms, axis=0)
    mom = jnp.stack(moms, axis=0)
    cnt = jnp.round(mom[:, 0])
    s = mom[:, 1] + mom[:, 2]
    ss = mom[:, 3] + mom[:, 4] + mom[:, 5]
    delta = jnp.maximum(2.0 * ss - s * s, 0.0)
    rmin = jnp.round(jnp.where(cnt > 1.5, 0.5 * (s - jnp.sqrt(delta)), s))
    r2nd = s - rmin
    cntc = jnp.minimum(cnt, 8.0)
    cums = [cntc[0:1]]
    for j in range(1, K_TOP):
        cums.append(cums[-1] + cntc[j:j + 1])
    cum = jnp.concatenate(cums, axis=0)
    prev = cum - cntc
    vs, ids = [], []
    for t in range(K_TOP):
        sel = (prev <= t) & (cum > t)
        vs.append(jnp.min(jnp.where(sel, mv, jnp.inf), axis=0, keepdims=True))
        rowpick = jnp.where(prev > t - 0.5, rmin, r2nd)
        ids.append(jnp.min(jnp.where(sel, rowpick, jnp.inf), axis=0,
                           keepdims=True))
    v_ref[...] = jnp.concatenate(vs, axis=0)
    i_ref[...] = (jnp.concatenate(ids, axis=0)
                  + (ki * tk).astype(jnp.float32)).astype(jnp.int32)


def _p2_body(v_ref, i_ref, qi_ref, kf_ref, td_ref, *, tq2):
    qb = pl.program_id(0)
    v = v_ref[...]
    idx = i_ref[...]
    tds, kfs = [], []
    for _ in range(K_TOP):
        m = jnp.min(v, axis=0, keepdims=True)
        am = jnp.min(jnp.where(v == m, idx, BIG_I), axis=0,
                     keepdims=True)
        tds.append(m)
        kfs.append(am)
        v = jnp.where(idx == am, jnp.inf, v)
    d2t = jnp.concatenate(tds, axis=0)
    kf = jnp.concatenate(kfs, axis=0)
    td = jnp.sqrt(jnp.maximum(d2t, 1e-12))
    valid = td <= MAP_RES
    rows = qb * tq2 + jax.lax.broadcasted_iota(jnp.int32, td.shape, 1)
    qi_ref[...] = jnp.where(valid, rows, -1)
    kf_ref[...] = jnp.where(valid, kf, -1)
    td_ref[...] = td


def kernel(queries, keys, k):
    del k
    nq, d = queries.shape
    k_real = keys.shape[0]
    kp = ((k_real + TK - 1) // TK) * TK
    nb = kp // TK
    keys_p = jnp.pad(keys, ((0, kp - k_real), (0, 0)))

    p1 = pl.pallas_call(
        functools.partial(_p1_body, k_real=k_real, tk=TK),
        grid=(nb, nq // TQ),
        in_specs=[
            pl.BlockSpec((TK, d), lambda ki, qi: (ki, 0)),
            pl.BlockSpec((TQ, d), lambda ki, qi: (qi, 0)),
        ],
        out_specs=[
            pl.BlockSpec((K_TOP, TQ), lambda ki, qi: (ki, qi)),
            pl.BlockSpec((K_TOP, TQ), lambda ki, qi: (ki, qi)),
        ],
        out_shape=[
            jax.ShapeDtypeStruct((nb * K_TOP, nq), jnp.float32),
            jax.ShapeDtypeStruct((nb * K_TOP, nq), jnp.int32),
        ],
    )
    pv, pi = p1(keys_p, queries)

    p2 = pl.pallas_call(
        functools.partial(_p2_body, tq2=TQ2),
        grid=(nq // TQ2,),
        in_specs=[
            pl.BlockSpec((nb * K_TOP, TQ2), lambda qb: (0, qb)),
            pl.BlockSpec((nb * K_TOP, TQ2), lambda qb: (0, qb)),
        ],
        out_specs=[
            pl.BlockSpec((K_TOP, TQ2), lambda qb: (0, qb)),
            pl.BlockSpec((K_TOP, TQ2), lambda qb: (0, qb)),
            pl.BlockSpec((K_TOP, TQ2), lambda qb: (0, qb)),
        ],
        out_shape=[
            jax.ShapeDtypeStruct((K_TOP, nq), jnp.int32),
            jax.ShapeDtypeStruct((K_TOP, nq), jnp.int32),
            jax.ShapeDtypeStruct((K_TOP, nq), jnp.float32),
        ],
    )
    qi_t, kf_t, td_t = p2(pv, pi)
    return qi_t.T, kf_t.T, td_t.T


# speedup vs baseline: 4.7958x; 1.0380x over previous
import functools

import jax
import jax.numpy as jnp
from jax.experimental import pallas as pl

MAP_RES = 16.0
K_TOP = 8
TQ = 512
TK = 2048
TQ2 = 512
BIG_I = 2**30


def _p1_body(k_ref, q_ref, qn_ref, kn_ref, v_ref, i_ref, *, tk):
    ki = pl.program_id(0)
    kb = k_ref[...]
    q = q_ref[...]
    qn = qn_ref[0]
    kn = kn_ref[0]
    dot = jax.lax.dot_general(kb, q, (((1,), (1,)), ((), ())),
                              preferred_element_type=jnp.float32)
    d2 = (qn + kn) - 2.0 * dot
    r1 = jax.lax.broadcasted_iota(jnp.int32, (1, tk), 1)
    r2i = r1 * r1
    w = jnp.concatenate([
        jnp.ones((1, tk), jnp.int32),
        (r1 >> 3) << 3, r1 & 7,
        (r2i >> 16) << 16, ((r2i >> 8) & 255) << 8, r2i & 255,
    ], axis=0).astype(jnp.float32)
    ms, moms = [], []
    for _ in range(K_TOP):
        m = jnp.min(d2, axis=0, keepdims=True)
        bits = d2 == m
        onehot = jnp.where(bits, 1.0, 0.0)
        moms.append(jax.lax.dot_general(w, onehot, (((1,), (0,)), ((), ())),
                                        preferred_element_type=jnp.float32))
        ms.append(m)
        d2 = jnp.where(bits, jnp.inf, d2)
    mv = jnp.concatenate(ms, axis=0)
    mom = jnp.stack(moms, axis=0)
    cnt = jnp.round(mom[:, 0])
    s = mom[:, 1] + mom[:, 2]
    ss = mom[:, 3] + mom[:, 4] + mom[:, 5]
    delta = jnp.maximum(2.0 * ss - s * s, 0.0)
    rmin = jnp.round(jnp.where(cnt > 1.5, 0.5 * (s - jnp.sqrt(delta)), s))
    r2nd = s - rmin
    cntc = jnp.minimum(cnt, 8.0)
    cums = [cntc[0:1]]
    for j in range(1, K_TOP):
        cums.append(cums[-1] + cntc[j:j + 1])
    cum = jnp.concatenate(cums, axis=0)
    prev = cum - cntc
    vs, ids = [], []
    for t in range(K_TOP):
        sel = (prev <= t) & (cum > t)
        vs.append(jnp.min(jnp.where(sel, mv, jnp.inf), axis=0, keepdims=True))
        rowpick = jnp.where(prev > t - 0.5, rmin, r2nd)
        ids.append(jnp.min(jnp.where(sel, rowpick, jnp.inf), axis=0,
                           keepdims=True))
    v_ref[...] = jnp.concatenate(vs, axis=0)
    i_ref[...] = (jnp.concatenate(ids, axis=0)
                  + (ki * tk).astype(jnp.float32)).astype(jnp.int32)


def _p2_body(v_ref, i_ref, qi_ref, kf_ref, td_ref, *, tq2):
    qb = pl.program_id(0)
    v = v_ref[...]
    idx = i_ref[...]
    tds, kfs = [], []
    for _ in range(K_TOP):
        m = jnp.min(v, axis=0, keepdims=True)
        am = jnp.min(jnp.where(v == m, idx, BIG_I), axis=0,
                     keepdims=True)
        tds.append(m)
        kfs.append(am)
        v = jnp.where(idx == am, jnp.inf, v)
    d2t = jnp.concatenate(tds, axis=0)
    kf = jnp.concatenate(kfs, axis=0)
    td = jnp.sqrt(jnp.maximum(d2t, 1e-12))
    valid = td <= MAP_RES
    rows = qb * tq2 + jax.lax.broadcasted_iota(jnp.int32, td.shape, 1)
    qi_ref[...] = jnp.where(valid, rows, -1)
    kf_ref[...] = jnp.where(valid, kf, -1)
    td_ref[...] = td


def kernel(queries, keys, k):
    del k
    nq, d = queries.shape
    k_real = keys.shape[0]
    kp = ((k_real + TK - 1) // TK) * TK
    nb = kp // TK
    keys_p = jnp.pad(keys, ((0, kp - k_real), (0, 0)), constant_values=1e6)
    qn_in = jnp.sum(queries * queries, axis=-1).reshape(nq // TQ, 1, TQ)
    kn_in = jnp.sum(keys_p * keys_p, axis=-1).reshape(nb, TK, 1)

    p1 = pl.pallas_call(
        functools.partial(_p1_body, tk=TK),
        grid=(nb, nq // TQ),
        in_specs=[
            pl.BlockSpec((TK, d), lambda ki, qi: (ki, 0)),
            pl.BlockSpec((TQ, d), lambda ki, qi: (qi, 0)),
            pl.BlockSpec((1, 1, TQ), lambda ki, qi: (qi, 0, 0)),
            pl.BlockSpec((1, TK, 1), lambda ki, qi: (ki, 0, 0)),
        ],
        out_specs=[
            pl.BlockSpec((K_TOP, TQ), lambda ki, qi: (ki, qi)),
            pl.BlockSpec((K_TOP, TQ), lambda ki, qi: (ki, qi)),
        ],
        out_shape=[
            jax.ShapeDtypeStruct((nb * K_TOP, nq), jnp.float32),
            jax.ShapeDtypeStruct((nb * K_TOP, nq), jnp.int32),
        ],
    )
    pv, pi = p1(keys_p, queries, qn_in, kn_in)

    p2 = pl.pallas_call(
        functools.partial(_p2_body, tq2=TQ2),
        grid=(nq // TQ2,),
        in_specs=[
            pl.BlockSpec((nb * K_TOP, TQ2), lambda qb: (0, qb)),
            pl.BlockSpec((nb * K_TOP, TQ2), lambda qb: (0, qb)),
        ],
        out_specs=[
            pl.BlockSpec((K_TOP, TQ2), lambda qb: (0, qb)),
            pl.BlockSpec((K_TOP, TQ2), lambda qb: (0, qb)),
            pl.BlockSpec((K_TOP, TQ2), lambda qb: (0, qb)),
        ],
        out_shape=[
            jax.ShapeDtypeStruct((K_TOP, nq), jnp.int32),
            jax.ShapeDtypeStruct((K_TOP, nq), jnp.int32),
            jax.ShapeDtypeStruct((K_TOP, nq), jnp.float32),
        ],
    )
    qi_t, kf_t, td_t = p2(pv, pi)
    return qi_t.T, kf_t.T, td_t.T
